# baseline (speedup 1.0000x reference)
import functools

import jax
import jax.numpy as jnp
from jax import lax
from jax.experimental import pallas as pl
from jax.experimental.pallas import tpu as pltpu
from jax.experimental.pallas import tpu_sc as plsc

S = 2048
H = 768
F = 2048
E = 64
TOPK = 2
A = S * TOPK
TROW = 128
MAX_TILES = 96
MAX_ROWS = MAX_TILES * TROW
FB = 512

_NC = 2
_NS = 16
_NW = _NC * _NS


def _router_body(x_ref, wg_ref, lg_ref, sel_ref, wt_ref):
    lg = lax.dot_general(x_ref[...], wg_ref[...], (((1,), (1,)), ((), ())),
                         preferred_element_type=jnp.float32)
    lg_ref[...] = lg
    iota = lax.broadcasted_iota(jnp.int32, lg.shape, 1)
    m1 = jnp.max(lg, axis=1, keepdims=True)
    e1 = jnp.min(jnp.where(lg == m1, iota, E), axis=1, keepdims=True)
    lg2 = jnp.where(iota == e1, -jnp.inf, lg)
    m2 = jnp.max(lg2, axis=1, keepdims=True)
    e2 = jnp.min(jnp.where(lg2 == m2, iota, E), axis=1, keepdims=True)
    w1 = 1.0 / (1.0 + jnp.exp(m2 - m1))
    sel_ref[...] = jnp.concatenate([e1, e2], axis=1)
    wt_ref[...] = jnp.concatenate([w1, 1.0 - w1], axis=1)


def _router(x, wg):
    sblk = 256
    return pl.pallas_call(
        _router_body,
        grid=(S // sblk,),
        in_specs=[
            pl.BlockSpec((sblk, H), lambda i: (i, 0)),
            pl.BlockSpec((E, H), lambda i: (0, 0)),
        ],
        out_specs=[
            pl.BlockSpec((sblk, E), lambda i: (i, 0)),
            pl.BlockSpec((sblk, TOPK), lambda i: (i, 0)),
            pl.BlockSpec((sblk, TOPK), lambda i: (i, 0)),
        ],
        out_shape=[
            jax.ShapeDtypeStruct((S, E), jnp.float32),
            jax.ShapeDtypeStruct((S, TOPK), jnp.int32),
            jax.ShapeDtypeStruct((S, TOPK), jnp.float32),
        ],
    )(x, wg)


_GW = 128
_CHUNKS = MAX_ROWS // (_NW * _GW)


def _sc_gather(x, token_ids):
    @functools.partial(
        pl.kernel,
        out_type=jax.ShapeDtypeStruct((MAX_ROWS, H), jnp.float32),
        mesh=plsc.VectorSubcoreMesh(core_axis_name="c", subcore_axis_name="s"),
        scratch_types=[
            pltpu.VMEM((_GW,), jnp.int32),
            pltpu.VMEM((_GW, H), jnp.float32),
            pltpu.SemaphoreType.DMA,
        ],
    )
    def k(x_hbm, ids_hbm, out_hbm, idx_v, rows_v, sem):
        wid = lax.axis_index("s") * _NC + lax.axis_index("c")

        @pl.loop(0, _CHUNKS)
        def _(c):
            base = wid * (_CHUNKS * _GW) + c * _GW
            pltpu.sync_copy(ids_hbm.at[pl.ds(base, _GW)], idx_v)
            pltpu.async_copy(x_hbm.at[idx_v], rows_v, sem).wait()
            pltpu.sync_copy(rows_v, out_hbm.at[pl.ds(base, _GW)])

    return k(x, token_ids)


def _mlp_body(te_ref, nr_ref, xg_ref, w_ref, w1_ref, w3_ref, w2_ref, y_ref):
    i = pl.program_id(0)
    f = pl.program_id(1)

    @pl.when(i < nr_ref[0])
    def _():
        xb = xg_ref[...].astype(jnp.bfloat16)
        a = lax.dot_general(xb, w1_ref[0].astype(jnp.bfloat16),
                            (((1,), (1,)), ((), ())),
                            preferred_element_type=jnp.float32)
        b = lax.dot_general(xb, w3_ref[0].astype(jnp.bfloat16),
                            (((1,), (1,)), ((), ())),
                            preferred_element_type=jnp.float32)
        h = (a * jax.nn.sigmoid(a)) * b
        yp = lax.dot_general(h.astype(jnp.bfloat16),
                             w2_ref[0].astype(jnp.bfloat16),
                             (((1,), (1,)), ((), ())),
                             preferred_element_type=jnp.float32)
        yp = yp * w_ref[...]

        @pl.when(f == 0)
        def _():
            y_ref[...] = yp

        @pl.when(f > 0)
        def _():
            y_ref[...] = y_ref[...] + yp


def _moe_mlp(xg, wpad, w1, w3, w2, tile_expert, n_real):
    grid_spec = pltpu.PrefetchScalarGridSpec(
        num_scalar_prefetch=2,
        grid=(MAX_TILES, F // FB),
        in_specs=[
            pl.BlockSpec((TROW, H), lambda i, f, te, nr: (i, 0)),
            pl.BlockSpec((TROW, 1), lambda i, f, te, nr: (i, 0)),
            pl.BlockSpec((1, FB, H), lambda i, f, te, nr: (te[i], f, 0)),
            pl.BlockSpec((1, FB, H), lambda i, f, te, nr: (te[i], f, 0)),
            pl.BlockSpec((1, H, FB), lambda i, f, te, nr: (te[i], 0, f)),
        ],
        out_specs=pl.BlockSpec((TROW, H), lambda i, f, te, nr: (i, 0)),
    )
    return pl.pallas_call(
        _mlp_body,
        grid_spec=grid_spec,
        out_shape=jax.ShapeDtypeStruct((MAX_ROWS, H), jnp.float32),
        compiler_params=pltpu.CompilerParams(
            dimension_semantics=("arbitrary", "arbitrary")),
    )(tile_expert, n_real, xg, wpad, w1, w3, w2)


_TPW = S // _NW


def _sc_combine(y, pos0, pos1):
    @functools.partial(
        pl.kernel,
        out_type=jax.ShapeDtypeStruct((S, H), jnp.float32),
        mesh=plsc.VectorSubcoreMesh(core_axis_name="c", subcore_axis_name="s"),
        scratch_types=[
            pltpu.VMEM((_TPW,), jnp.int32),
            pltpu.VMEM((_TPW,), jnp.int32),
            pltpu.VMEM((_TPW, H), jnp.float32),
            pltpu.VMEM((_TPW, H), jnp.float32),
            pltpu.SemaphoreType.DMA,
        ],
    )
    def k(y_hbm, p0_hbm, p1_hbm, out_hbm, i0, i1, b0, b1, sem):
        wid = lax.axis_index("s") * _NC + lax.axis_index("c")
        base = wid * _TPW
        pltpu.sync_copy(p0_hbm.at[pl.ds(base, _TPW)], i0)
        pltpu.sync_copy(p1_hbm.at[pl.ds(base, _TPW)], i1)
        pltpu.async_copy(y_hbm.at[i0], b0, sem).wait()
        pltpu.async_copy(y_hbm.at[i1], b1, sem).wait()

        @pl.loop(0, _TPW)
        def _(r):
            @pl.loop(0, H, step=16)
            def _(c):
                slc = (pl.ds(r, 1), pl.ds(c, 16))
                b0[slc] = b0[slc] + b1[slc]

        pltpu.sync_copy(b0, out_hbm.at[pl.ds(base, _TPW)])

    return k(y, pos0, pos1)


def kernel(hidden_states, Wg, W1, W3, W2):
    x = hidden_states.reshape(S, H)
    logits, sel, wts = _router(x, Wg)

    e_flat = sel.reshape(A)
    w_flat = wts.reshape(A)
    onehot = jax.nn.one_hot(e_flat, E, dtype=jnp.int32)
    cum = jnp.cumsum(onehot, axis=0)
    counts = cum[-1]
    rank = jnp.take_along_axis(cum, e_flat[:, None], axis=1)[:, 0] - 1
    n_tiles = (counts + TROW - 1) // TROW
    cum_tiles = jnp.cumsum(n_tiles)
    row_base = (cum_tiles - n_tiles) * TROW
    prow = (row_base[e_flat] + rank).astype(jnp.int32)
    n_real = cum_tiles[-1:].astype(jnp.int32)
    tile_expert = jnp.minimum(
        jnp.searchsorted(cum_tiles, jnp.arange(MAX_TILES), side="right"),
        E - 1).astype(jnp.int32)

    token_ids = (jnp.arange(MAX_ROWS, dtype=jnp.int32) % S).at[prow].set(
        jnp.arange(A, dtype=jnp.int32) // TOPK)
    wpad = jnp.zeros((MAX_ROWS, 1), jnp.float32).at[prow, 0].set(w_flat)
    pos0 = prow[0::2]
    pos1 = prow[1::2]

    xg = _sc_gather(x, token_ids)
    y = _moe_mlp(xg, wpad, W1, W3, W2, tile_expert, n_real)
    final = _sc_combine(y, pos0, pos1)
    return final.reshape(hidden_states.shape), logits


# speedup vs baseline: 1.2398x; 1.2398x over previous
import functools

import jax
import jax.numpy as jnp
from jax import lax
from jax.experimental import pallas as pl
from jax.experimental.pallas import tpu as pltpu
from jax.experimental.pallas import tpu_sc as plsc

S = 2048
H = 768
F = 2048
E = 64
TOPK = 2
A = S * TOPK
TROW = 128
MAX_TILES = 96
MAX_ROWS = MAX_TILES * TROW
FB = 512

_NC = 2
_NS = 16
_NW = _NC * _NS


def _router_body(x_ref, wg_ref, lg_ref, sel_ref, wt_ref):
    lg = lax.dot_general(x_ref[...], wg_ref[...], (((1,), (1,)), ((), ())),
                         preferred_element_type=jnp.float32)
    lg_ref[...] = lg
    iota = lax.broadcasted_iota(jnp.int32, lg.shape, 1)
    m1 = jnp.max(lg, axis=1, keepdims=True)
    e1 = jnp.min(jnp.where(lg == m1, iota, E), axis=1, keepdims=True)
    lg2 = jnp.where(iota == e1, -jnp.inf, lg)
    m2 = jnp.max(lg2, axis=1, keepdims=True)
    e2 = jnp.min(jnp.where(lg2 == m2, iota, E), axis=1, keepdims=True)
    w1 = 1.0 / (1.0 + jnp.exp(m2 - m1))
    sel_ref[...] = jnp.concatenate([e1, e2], axis=1)
    wt_ref[...] = jnp.concatenate([w1, 1.0 - w1], axis=1)


def _router(x, wg):
    sblk = 256
    return pl.pallas_call(
        _router_body,
        grid=(S // sblk,),
        in_specs=[
            pl.BlockSpec((sblk, H), lambda i: (i, 0)),
            pl.BlockSpec((E, H), lambda i: (0, 0)),
        ],
        out_specs=[
            pl.BlockSpec((sblk, E), lambda i: (i, 0)),
            pl.BlockSpec((sblk, TOPK), lambda i: (i, 0)),
            pl.BlockSpec((sblk, TOPK), lambda i: (i, 0)),
        ],
        out_shape=[
            jax.ShapeDtypeStruct((S, E), jnp.float32),
            jax.ShapeDtypeStruct((S, TOPK), jnp.int32),
            jax.ShapeDtypeStruct((S, TOPK), jnp.float32),
        ],
    )(x, wg)


_GW = 128
_CHUNKS = MAX_ROWS // (_NW * _GW)


def _sc_gather(x, token_ids):
    @functools.partial(
        pl.kernel,
        out_type=jax.ShapeDtypeStruct((MAX_ROWS, H), jnp.float32),
        mesh=plsc.VectorSubcoreMesh(core_axis_name="c", subcore_axis_name="s"),
        scratch_types=[
            pltpu.VMEM((_GW,), jnp.int32),
            pltpu.VMEM((_GW, H), jnp.float32),
            pltpu.SemaphoreType.DMA,
        ],
    )
    def k(x_hbm, ids_hbm, out_hbm, idx_v, rows_v, sem):
        wid = lax.axis_index("s") * _NC + lax.axis_index("c")

        @pl.loop(0, _CHUNKS)
        def _(c):
            base = wid * (_CHUNKS * _GW) + c * _GW
            pltpu.sync_copy(ids_hbm.at[pl.ds(base, _GW)], idx_v)
            pltpu.async_copy(x_hbm.at[idx_v], rows_v, sem).wait()
            pltpu.sync_copy(rows_v, out_hbm.at[pl.ds(base, _GW)])

    return k(x, token_ids)


def _mlp_body(te_ref, nr_ref, xg_ref, w_ref, w1_ref, w3_ref, w2_ref, y_ref):
    i = pl.program_id(0)
    f = pl.program_id(1)

    @pl.when(i < nr_ref[0])
    def _():
        xb = xg_ref[...].astype(jnp.bfloat16)
        a = lax.dot_general(xb, w1_ref[0].astype(jnp.bfloat16),
                            (((1,), (1,)), ((), ())),
                            preferred_element_type=jnp.float32)
        b = lax.dot_general(xb, w3_ref[0].astype(jnp.bfloat16),
                            (((1,), (1,)), ((), ())),
                            preferred_element_type=jnp.float32)
        h = (a * jax.nn.sigmoid(a)) * b
        yp = lax.dot_general(h.astype(jnp.bfloat16),
                             w2_ref[0].astype(jnp.bfloat16),
                             (((1,), (1,)), ((), ())),
                             preferred_element_type=jnp.float32)
        yp = yp * w_ref[...]

        @pl.when(f == 0)
        def _():
            y_ref[...] = yp

        @pl.when(f > 0)
        def _():
            y_ref[...] = y_ref[...] + yp


def _moe_mlp(xg, wpad, w1, w3, w2, tile_expert, n_real):
    grid_spec = pltpu.PrefetchScalarGridSpec(
        num_scalar_prefetch=2,
        grid=(MAX_TILES, F // FB),
        in_specs=[
            pl.BlockSpec((TROW, H), lambda i, f, te, nr: (i, 0)),
            pl.BlockSpec((TROW, 1), lambda i, f, te, nr: (i, 0)),
            pl.BlockSpec((1, FB, H),
                         lambda i, f, te, nr: (te[i], jnp.where(i < nr[0], f, 0), 0)),
            pl.BlockSpec((1, FB, H),
                         lambda i, f, te, nr: (te[i], jnp.where(i < nr[0], f, 0), 0)),
            pl.BlockSpec((1, H, FB),
                         lambda i, f, te, nr: (te[i], 0, jnp.where(i < nr[0], f, 0))),
        ],
        out_specs=pl.BlockSpec((TROW, H), lambda i, f, te, nr: (i, 0)),
    )
    return pl.pallas_call(
        _mlp_body,
        grid_spec=grid_spec,
        out_shape=jax.ShapeDtypeStruct((MAX_ROWS, H), jnp.float32),
        compiler_params=pltpu.CompilerParams(
            dimension_semantics=("arbitrary", "arbitrary")),
    )(tile_expert, n_real, xg, wpad, w1, w3, w2)


_TPW = S // _NW


def _sc_combine(y, pos0, pos1):
    @functools.partial(
        pl.kernel,
        out_type=jax.ShapeDtypeStruct((S, H), jnp.float32),
        mesh=plsc.VectorSubcoreMesh(core_axis_name="c", subcore_axis_name="s"),
        scratch_types=[
            pltpu.VMEM((_TPW,), jnp.int32),
            pltpu.VMEM((_TPW,), jnp.int32),
            pltpu.VMEM((_TPW, H), jnp.float32),
            pltpu.VMEM((_TPW, H), jnp.float32),
            pltpu.SemaphoreType.DMA,
        ],
    )
    def k(y_hbm, p0_hbm, p1_hbm, out_hbm, i0, i1, b0, b1, sem):
        wid = lax.axis_index("s") * _NC + lax.axis_index("c")
        base = wid * _TPW
        pltpu.sync_copy(p0_hbm.at[pl.ds(base, _TPW)], i0)
        pltpu.sync_copy(p1_hbm.at[pl.ds(base, _TPW)], i1)
        pltpu.async_copy(y_hbm.at[i0], b0, sem).wait()
        pltpu.async_copy(y_hbm.at[i1], b1, sem).wait()

        @pl.loop(0, _TPW)
        def _(r):
            @pl.loop(0, H, step=16)
            def _(c):
                slc = (pl.ds(r, 1), pl.ds(c, 16))
                b0[slc] = b0[slc] + b1[slc]

        pltpu.sync_copy(b0, out_hbm.at[pl.ds(base, _TPW)])

    return k(y, pos0, pos1)


def kernel(hidden_states, Wg, W1, W3, W2):
    x = hidden_states.reshape(S, H)
    logits, sel, wts = _router(x, Wg)

    e_flat = sel.reshape(A)
    w_flat = wts.reshape(A)
    onehot = jax.nn.one_hot(e_flat, E, dtype=jnp.int32)
    cum = jnp.cumsum(onehot, axis=0)
    counts = cum[-1]
    rank = jnp.take_along_axis(cum, e_flat[:, None], axis=1)[:, 0] - 1
    n_tiles = (counts + TROW - 1) // TROW
    cum_tiles = jnp.cumsum(n_tiles)
    row_base = (cum_tiles - n_tiles) * TROW
    prow = (row_base[e_flat] + rank).astype(jnp.int32)
    n_real = cum_tiles[-1:].astype(jnp.int32)
    tile_expert = jnp.minimum(
        jnp.searchsorted(cum_tiles, jnp.arange(MAX_TILES), side="right"),
        E - 1).astype(jnp.int32)

    token_ids = (jnp.arange(MAX_ROWS, dtype=jnp.int32) % S).at[prow].set(
        jnp.arange(A, dtype=jnp.int32) // TOPK)
    wpad = jnp.zeros((MAX_ROWS, 1), jnp.float32).at[prow, 0].set(w_flat)
    pos0 = prow[0::2]
    pos1 = prow[1::2]

    xg = _sc_gather(x, token_ids)
    y = _moe_mlp(xg, wpad, W1, W3, W2, tile_expert, n_real)
    final = _sc_combine(y, pos0, pos1)
    return final.reshape(hidden_states.shape), logits


# speedup vs baseline: 1.5111x; 1.2188x over previous
import functools

import jax
import jax.numpy as jnp
from jax import lax
from jax.experimental import pallas as pl
from jax.experimental.pallas import tpu as pltpu
from jax.experimental.pallas import tpu_sc as plsc

S = 2048
H = 768
F = 2048
E = 64
TOPK = 2
A = S * TOPK
TROW = 128
MAX_TILES = 96
MAX_ROWS = MAX_TILES * TROW
FB = 512

_NC = 2
_NS = 16
_NW = _NC * _NS


def _router_body(x_ref, wg_ref, lg_ref, sel_ref, wt_ref):
    lg = lax.dot_general(x_ref[...], wg_ref[...], (((1,), (1,)), ((), ())),
                         preferred_element_type=jnp.float32)
    lg_ref[...] = lg
    iota = lax.broadcasted_iota(jnp.int32, lg.shape, 1)
    m1 = jnp.max(lg, axis=1, keepdims=True)
    e1 = jnp.min(jnp.where(lg == m1, iota, E), axis=1, keepdims=True)
    lg2 = jnp.where(iota == e1, -jnp.inf, lg)
    m2 = jnp.max(lg2, axis=1, keepdims=True)
    e2 = jnp.min(jnp.where(lg2 == m2, iota, E), axis=1, keepdims=True)
    w1 = 1.0 / (1.0 + jnp.exp(m2 - m1))
    sel_ref[...] = jnp.concatenate([e1, e2], axis=1)
    wt_ref[...] = jnp.concatenate([w1, 1.0 - w1], axis=1)


def _router(x, wg):
    sblk = 256
    return pl.pallas_call(
        _router_body,
        grid=(S // sblk,),
        in_specs=[
            pl.BlockSpec((sblk, H), lambda i: (i, 0)),
            pl.BlockSpec((E, H), lambda i: (0, 0)),
        ],
        out_specs=[
            pl.BlockSpec((sblk, E), lambda i: (i, 0)),
            pl.BlockSpec((sblk, TOPK), lambda i: (i, 0)),
            pl.BlockSpec((sblk, TOPK), lambda i: (i, 0)),
        ],
        out_shape=[
            jax.ShapeDtypeStruct((S, E), jnp.float32),
            jax.ShapeDtypeStruct((S, TOPK), jnp.int32),
            jax.ShapeDtypeStruct((S, TOPK), jnp.float32),
        ],
    )(x, wg)


_GW = 128
_CHUNKS = MAX_ROWS // (_NW * _GW)


def _sc_gather(x, token_ids):
    @functools.partial(
        pl.kernel,
        out_type=jax.ShapeDtypeStruct((MAX_ROWS, H), jnp.float32),
        mesh=plsc.VectorSubcoreMesh(core_axis_name="c", subcore_axis_name="s"),
        scratch_types=[
            pltpu.VMEM((_GW,), jnp.int32),
            pltpu.VMEM((_GW, H), jnp.float32),
            pltpu.SemaphoreType.DMA,
        ],
    )
    def k(x_hbm, ids_hbm, out_hbm, idx_v, rows_v, sem):
        wid = lax.axis_index("s") * _NC + lax.axis_index("c")

        @pl.loop(0, _CHUNKS)
        def _(c):
            base = wid * (_CHUNKS * _GW) + c * _GW
            pltpu.sync_copy(ids_hbm.at[pl.ds(base, _GW)], idx_v)
            pltpu.async_copy(x_hbm.at[idx_v], rows_v, sem).wait()
            pltpu.sync_copy(rows_v, out_hbm.at[pl.ds(base, _GW)])

    return k(x, token_ids)


def _mlp_body(te_ref, nr_ref, xg_ref, w_ref, w1_ref, w3_ref, w2_ref, y_ref):
    i = pl.program_id(0)

    @pl.when(i < nr_ref[0])
    def _():
        xb = xg_ref[...].astype(jnp.bfloat16)
        a = lax.dot_general(xb, w1_ref[0].astype(jnp.bfloat16),
                            (((1,), (1,)), ((), ())),
                            preferred_element_type=jnp.float32)
        b = lax.dot_general(xb, w3_ref[0].astype(jnp.bfloat16),
                            (((1,), (1,)), ((), ())),
                            preferred_element_type=jnp.float32)
        h = (a * jax.nn.sigmoid(a)) * b
        yp = lax.dot_general(h.astype(jnp.bfloat16),
                             w2_ref[0].astype(jnp.bfloat16),
                             (((1,), (1,)), ((), ())),
                             preferred_element_type=jnp.float32)
        y_ref[...] = yp * w_ref[...]


def _moe_mlp(xg, wpad, w1, w3, w2, tile_expert, n_real):
    def we_idx(i, te, nr):
        return jnp.minimum(te[i], te[jnp.minimum(nr[0] - 1, MAX_TILES - 1)])

    grid_spec = pltpu.PrefetchScalarGridSpec(
        num_scalar_prefetch=2,
        grid=(MAX_TILES,),
        in_specs=[
            pl.BlockSpec((TROW, H), lambda i, te, nr: (i, 0)),
            pl.BlockSpec((TROW, 1), lambda i, te, nr: (i, 0)),
            pl.BlockSpec((1, F, H), lambda i, te, nr: (we_idx(i, te, nr), 0, 0)),
            pl.BlockSpec((1, F, H), lambda i, te, nr: (we_idx(i, te, nr), 0, 0)),
            pl.BlockSpec((1, H, F), lambda i, te, nr: (we_idx(i, te, nr), 0, 0)),
        ],
        out_specs=pl.BlockSpec((TROW, H), lambda i, te, nr: (i, 0)),
    )
    return pl.pallas_call(
        _mlp_body,
        grid_spec=grid_spec,
        out_shape=jax.ShapeDtypeStruct((MAX_ROWS, H), jnp.float32),
        compiler_params=pltpu.CompilerParams(
            dimension_semantics=("arbitrary",)),
    )(tile_expert, n_real, xg, wpad, w1, w3, w2)


_TPW = S // _NW


def _sc_combine(y, pos0, pos1):
    @functools.partial(
        pl.kernel,
        out_type=jax.ShapeDtypeStruct((S, H), jnp.float32),
        mesh=plsc.VectorSubcoreMesh(core_axis_name="c", subcore_axis_name="s"),
        scratch_types=[
            pltpu.VMEM((_TPW,), jnp.int32),
            pltpu.VMEM((_TPW,), jnp.int32),
            pltpu.VMEM((_TPW, H), jnp.float32),
            pltpu.VMEM((_TPW, H), jnp.float32),
            pltpu.SemaphoreType.DMA,
        ],
    )
    def k(y_hbm, p0_hbm, p1_hbm, out_hbm, i0, i1, b0, b1, sem):
        wid = lax.axis_index("s") * _NC + lax.axis_index("c")
        base = wid * _TPW
        pltpu.sync_copy(p0_hbm.at[pl.ds(base, _TPW)], i0)
        pltpu.sync_copy(p1_hbm.at[pl.ds(base, _TPW)], i1)
        pltpu.async_copy(y_hbm.at[i0], b0, sem).wait()
        pltpu.async_copy(y_hbm.at[i1], b1, sem).wait()

        @pl.loop(0, _TPW)
        def _(r):
            @pl.loop(0, H, step=16)
            def _(c):
                slc = (pl.ds(r, 1), pl.ds(c, 16))
                b0[slc] = b0[slc] + b1[slc]

        pltpu.sync_copy(b0, out_hbm.at[pl.ds(base, _TPW)])

    return k(y, pos0, pos1)


def kernel(hidden_states, Wg, W1, W3, W2):
    x = hidden_states.reshape(S, H)
    logits, sel, wts = _router(x, Wg)

    e_flat = sel.reshape(A)
    w_flat = wts.reshape(A)
    onehot = jax.nn.one_hot(e_flat, E, dtype=jnp.int32)
    cum = jnp.cumsum(onehot, axis=0)
    counts = cum[-1]
    rank = jnp.take_along_axis(cum, e_flat[:, None], axis=1)[:, 0] - 1
    n_tiles = (counts + TROW - 1) // TROW
    cum_tiles = jnp.cumsum(n_tiles)
    row_base = (cum_tiles - n_tiles) * TROW
    prow = (row_base[e_flat] + rank).astype(jnp.int32)
    n_real = cum_tiles[-1:].astype(jnp.int32)
    tile_expert = jnp.minimum(
        jnp.searchsorted(cum_tiles, jnp.arange(MAX_TILES), side="right"),
        E - 1).astype(jnp.int32)

    token_ids = (jnp.arange(MAX_ROWS, dtype=jnp.int32) % S).at[prow].set(
        jnp.arange(A, dtype=jnp.int32) // TOPK)
    wpad = jnp.zeros((MAX_ROWS, 1), jnp.float32).at[prow, 0].set(w_flat)
    pos0 = prow[0::2]
    pos1 = prow[1::2]

    xg = _sc_gather(x, token_ids)
    y = _moe_mlp(xg, wpad, W1, W3, W2, tile_expert, n_real)
    final = _sc_combine(y, pos0, pos1)
    return final.reshape(hidden_states.shape), logits


# speedup vs baseline: 1.6734x; 1.1074x over previous
import dataclasses
import functools

import jax
import jax.numpy as jnp
from jax import lax
from jax.experimental import pallas as pl
from jax.experimental.pallas import tpu as pltpu
from jax.experimental.pallas import tpu_sc as plsc

S = 2048
H = 768
F = 2048
E = 64
TOPK = 2
A = S * TOPK
TROW = 128
MAX_TILES = 96
MAX_ROWS = MAX_TILES * TROW

_NC = 2
_NS = 16
_NW = _NC * _NS
_SBLK = 256


def _router_body(x_ref, wg_ref, lg_ref, sel_ref, wt_ref, rank_ref, cnt_out_ref,
                 cnt_ref):
    i = pl.program_id(0)

    @pl.when(i == 0)
    def _():
        cnt_ref[...] = jnp.zeros((1, E), jnp.float32)

    lg = lax.dot_general(x_ref[...], wg_ref[...], (((1,), (1,)), ((), ())),
                         preferred_element_type=jnp.float32)
    lg_ref[...] = lg
    iota = lax.broadcasted_iota(jnp.int32, lg.shape, 1)
    m1 = jnp.max(lg, axis=1, keepdims=True)
    e1 = jnp.min(jnp.where(lg == m1, iota, E), axis=1, keepdims=True)
    lg2 = jnp.where(iota == e1, -jnp.inf, lg)
    m2 = jnp.max(lg2, axis=1, keepdims=True)
    e2 = jnp.min(jnp.where(lg2 == m2, iota, E), axis=1, keepdims=True)
    w1 = 1.0 / (1.0 + jnp.exp(m2 - m1))
    sel_ref[...] = jnp.concatenate([e1, e2], axis=1)
    wt_ref[...] = jnp.concatenate([w1, 1.0 - w1], axis=1)

    oh0 = (iota == e1).astype(jnp.bfloat16)
    oh1 = (iota == e2).astype(jnp.bfloat16)
    rr = lax.broadcasted_iota(jnp.int32, (_SBLK, _SBLK), 0)
    cc = lax.broadcasted_iota(jnp.int32, (_SBLK, _SBLK), 1)
    ltri = (rr > cc).astype(jnp.bfloat16)
    p0 = lax.dot_general(ltri, oh0, (((1,), (0,)), ((), ())),
                         preferred_element_type=jnp.float32)
    p1 = lax.dot_general(ltri, oh1, (((1,), (0,)), ((), ())),
                         preferred_element_type=jnp.float32)
    oh0f = oh0.astype(jnp.float32)
    oh1f = oh1.astype(jnp.float32)
    s0 = jnp.sum(oh0f, axis=0, keepdims=True)
    cnt = cnt_ref[...]
    rank0 = jnp.sum(oh0f * (p0 + cnt), axis=1, keepdims=True)
    rank1 = jnp.sum(oh1f * (p1 + cnt + s0), axis=1, keepdims=True)
    rank_ref[...] = jnp.concatenate([rank0, rank1], axis=1).astype(jnp.int32)
    cnt_new = cnt + s0 + jnp.sum(oh1f, axis=0, keepdims=True)
    cnt_ref[...] = cnt_new

    @pl.when(i == (S // _SBLK) - 1)
    def _():
        cnt_out_ref[...] = cnt_new.astype(jnp.int32)


def _router(x, wg):
    return pl.pallas_call(
        _router_body,
        grid=(S // _SBLK,),
        in_specs=[
            pl.BlockSpec((_SBLK, H), lambda i: (i, 0)),
            pl.BlockSpec((E, H), lambda i: (0, 0)),
        ],
        out_specs=[
            pl.BlockSpec((_SBLK, E), lambda i: (i, 0)),
            pl.BlockSpec((_SBLK, TOPK), lambda i: (i, 0)),
            pl.BlockSpec((_SBLK, TOPK), lambda i: (i, 0)),
            pl.BlockSpec((_SBLK, TOPK), lambda i: (i, 0)),
            pl.BlockSpec((1, E), lambda i: (0, 0)),
        ],
        out_shape=[
            jax.ShapeDtypeStruct((S, E), jnp.float32),
            jax.ShapeDtypeStruct((S, TOPK), jnp.int32),
            jax.ShapeDtypeStruct((S, TOPK), jnp.float32),
            jax.ShapeDtypeStruct((S, TOPK), jnp.int32),
            jax.ShapeDtypeStruct((1, E), jnp.int32),
        ],
        scratch_shapes=[pltpu.VMEM((1, E), jnp.float32)],
    )(x, wg)


_APW = A // _NW


def _sc_stage(x, tok_flat, prow_flat):
    @functools.partial(
        pl.kernel,
        out_type=jax.ShapeDtypeStruct((MAX_ROWS, H), jnp.float32),
        mesh=plsc.VectorSubcoreMesh(core_axis_name="c", subcore_axis_name="s"),
        scratch_types=[
            pltpu.VMEM((_APW,), jnp.int32),
            pltpu.VMEM((_APW,), jnp.int32),
            pltpu.VMEM((_APW, H), jnp.float32),
            pltpu.SemaphoreType.DMA,
        ],
    )
    def k(x_hbm, tok_hbm, prow_hbm, xg_hbm, ti, pi, rows, sem):
        wid = lax.axis_index("s") * _NC + lax.axis_index("c")
        base = wid * _APW
        pltpu.sync_copy(tok_hbm.at[pl.ds(base, _APW)], ti)
        pltpu.sync_copy(prow_hbm.at[pl.ds(base, _APW)], pi)
        pltpu.async_copy(x_hbm.at[ti], rows, sem).wait()
        pltpu.sync_copy(rows, xg_hbm.at[pi])

    return k(x, tok_flat, prow_flat)


def _mlp_body(te_ref, nr_ref, xg_ref, w1_ref, w3_ref, w2_ref, y_ref):
    i = pl.program_id(0)

    @pl.when(i < nr_ref[0])
    def _():
        xb = xg_ref[...].astype(jnp.bfloat16)
        a = lax.dot_general(xb, w1_ref[0].astype(jnp.bfloat16),
                            (((1,), (1,)), ((), ())),
                            preferred_element_type=jnp.float32)
        b = lax.dot_general(xb, w3_ref[0].astype(jnp.bfloat16),
                            (((1,), (1,)), ((), ())),
                            preferred_element_type=jnp.float32)
        h = (a * jax.nn.sigmoid(a)) * b
        y_ref[...] = lax.dot_general(h.astype(jnp.bfloat16),
                                     w2_ref[0].astype(jnp.bfloat16),
                                     (((1,), (1,)), ((), ())),
                                     preferred_element_type=jnp.float32)


def _moe_mlp(xg, w1, w3, w2, tile_expert, n_real):
    def we_idx(i, te, nr):
        return jnp.minimum(te[i], te[jnp.minimum(nr[0] - 1, MAX_TILES - 1)])

    grid_spec = pltpu.PrefetchScalarGridSpec(
        num_scalar_prefetch=2,
        grid=(MAX_TILES,),
        in_specs=[
            pl.BlockSpec((TROW, H), lambda i, te, nr: (i, 0)),
            pl.BlockSpec((1, F, H), lambda i, te, nr: (we_idx(i, te, nr), 0, 0)),
            pl.BlockSpec((1, F, H), lambda i, te, nr: (we_idx(i, te, nr), 0, 0)),
            pl.BlockSpec((1, H, F), lambda i, te, nr: (we_idx(i, te, nr), 0, 0)),
        ],
        out_specs=pl.BlockSpec((TROW, H), lambda i, te, nr: (i, 0)),
    )
    return pl.pallas_call(
        _mlp_body,
        grid_spec=grid_spec,
        out_shape=jax.ShapeDtypeStruct((MAX_ROWS, H), jnp.float32),
        compiler_params=pltpu.CompilerParams(
            dimension_semantics=("arbitrary",)),
    )(tile_expert, n_real, xg, w1, w3, w2)


_TPW = S // _NW


def _sc_cparams():
    cp = pltpu.CompilerParams()
    if "needs_layout_passes" in pltpu.CompilerParams.__dataclass_fields__:
        cp = dataclasses.replace(cp, needs_layout_passes=False)
    return cp


def _sc_combine(y, pos0, pos1, w_flat):
    @functools.partial(
        pl.kernel,
        out_type=jax.ShapeDtypeStruct((S, H), jnp.float32),
        mesh=plsc.VectorSubcoreMesh(core_axis_name="c", subcore_axis_name="s"),
        compiler_params=_sc_cparams(),
        scratch_types=[
            pltpu.VMEM((_TPW,), jnp.int32),
            pltpu.VMEM((_TPW,), jnp.int32),
            pltpu.VMEM((2 * _TPW,), jnp.float32),
            pltpu.VMEM((_TPW, H), jnp.float32),
            pltpu.VMEM((_TPW, H), jnp.float32),
            pltpu.SemaphoreType.DMA,
        ],
    )
    def k(y_hbm, p0_hbm, p1_hbm, w_hbm, out_hbm, i0, i1, wv, b0, b1, sem):
        wid = lax.axis_index("s") * _NC + lax.axis_index("c")
        base = wid * _TPW
        pltpu.sync_copy(p0_hbm.at[pl.ds(base, _TPW)], i0)
        pltpu.sync_copy(p1_hbm.at[pl.ds(base, _TPW)], i1)
        pltpu.sync_copy(w_hbm.at[pl.ds(2 * base, 2 * _TPW)], wv)
        pltpu.async_copy(y_hbm.at[i0], b0, sem).wait()
        pltpu.async_copy(y_hbm.at[i1], b1, sem).wait()

        @pl.loop(0, _TPW)
        def _(r):
            w0 = plsc.load_gather(wv, [jnp.full((16,), 2 * r, jnp.int32)])
            w1 = plsc.load_gather(wv, [jnp.full((16,), 2 * r + 1, jnp.int32)])

            @pl.loop(0, H, step=16)
            def _(c):
                b0[r, pl.ds(c, 16)] = (b0[r, pl.ds(c, 16)] * w0 +
                                       b1[r, pl.ds(c, 16)] * w1)

        pltpu.sync_copy(b0, out_hbm.at[pl.ds(base, _TPW)])

    return k(y, pos0, pos1, w_flat)


def kernel(hidden_states, Wg, W1, W3, W2):
    x = hidden_states.reshape(S, H)
    logits, sel, wts, rank, counts = _router(x, Wg)

    counts = counts.reshape(E)
    n_tiles = (counts + TROW - 1) // TROW
    cum_tiles = jnp.cumsum(n_tiles)
    row_base = (cum_tiles - n_tiles) * TROW
    prow2 = (row_base[sel] + rank).astype(jnp.int32)
    n_real = cum_tiles[-1:].astype(jnp.int32)
    tile_expert = jnp.minimum(
        jnp.searchsorted(cum_tiles, jnp.arange(MAX_TILES), side="right"),
        E - 1).astype(jnp.int32)

    tok_flat = jnp.arange(A, dtype=jnp.int32) // TOPK
    prow_flat = prow2.reshape(A)
    w_flat = wts.reshape(A)

    xg = _sc_stage(x, tok_flat, prow_flat)
    y = _moe_mlp(xg, W1, W3, W2, tile_expert, n_real)
    final = _sc_combine(y, prow2[:, 0], prow2[:, 1], w_flat)
    return final.reshape(hidden_states.shape), logits


# speedup vs baseline: 1.7216x; 1.0288x over previous
import dataclasses
import functools

import jax
import jax.numpy as jnp
from jax import lax
from jax.experimental import pallas as pl
from jax.experimental.pallas import tpu as pltpu
from jax.experimental.pallas import tpu_sc as plsc

S = 2048
H = 768
F = 2048
E = 64
TOPK = 2
A = S * TOPK
TROW = 128
MAX_TILES = 96
MAX_ROWS = MAX_TILES * TROW

_NC = 2
_NS = 16
_NW = _NC * _NS
_SBLK = 256


def _router_body(x_ref, wg_ref, lg_ref, sel_ref, wt_ref, rank_ref, cnt_out_ref,
                 cnt_ref):
    i = pl.program_id(0)

    @pl.when(i == 0)
    def _():
        cnt_ref[...] = jnp.zeros((1, E), jnp.float32)

    lg = lax.dot_general(x_ref[...], wg_ref[...], (((1,), (1,)), ((), ())),
                         preferred_element_type=jnp.float32)
    lg_ref[...] = lg
    iota = lax.broadcasted_iota(jnp.int32, lg.shape, 1)
    m1 = jnp.max(lg, axis=1, keepdims=True)
    e1 = jnp.min(jnp.where(lg == m1, iota, E), axis=1, keepdims=True)
    lg2 = jnp.where(iota == e1, -jnp.inf, lg)
    m2 = jnp.max(lg2, axis=1, keepdims=True)
    e2 = jnp.min(jnp.where(lg2 == m2, iota, E), axis=1, keepdims=True)
    w1 = 1.0 / (1.0 + jnp.exp(m2 - m1))
    sel_ref[...] = jnp.concatenate([e1, e2], axis=1)
    wt_ref[...] = jnp.concatenate([w1, 1.0 - w1], axis=1)

    oh0 = (iota == e1).astype(jnp.bfloat16)
    oh1 = (iota == e2).astype(jnp.bfloat16)
    rr = lax.broadcasted_iota(jnp.int32, (_SBLK, _SBLK), 0)
    cc = lax.broadcasted_iota(jnp.int32, (_SBLK, _SBLK), 1)
    ltri = (rr > cc).astype(jnp.bfloat16)
    p0 = lax.dot_general(ltri, oh0, (((1,), (0,)), ((), ())),
                         preferred_element_type=jnp.float32)
    p1 = lax.dot_general(ltri, oh1, (((1,), (0,)), ((), ())),
                         preferred_element_type=jnp.float32)
    oh0f = oh0.astype(jnp.float32)
    oh1f = oh1.astype(jnp.float32)
    s0 = jnp.sum(oh0f, axis=0, keepdims=True)
    cnt = cnt_ref[...]
    rank0 = jnp.sum(oh0f * (p0 + cnt), axis=1, keepdims=True)
    rank1 = jnp.sum(oh1f * (p1 + cnt + s0), axis=1, keepdims=True)
    rank_ref[...] = jnp.concatenate([rank0, rank1], axis=1).astype(jnp.int32)
    cnt_new = cnt + s0 + jnp.sum(oh1f, axis=0, keepdims=True)
    cnt_ref[...] = cnt_new

    @pl.when(i == (S // _SBLK) - 1)
    def _():
        cnt_out_ref[...] = cnt_new.astype(jnp.int32)


def _router(x, wg):
    return pl.pallas_call(
        _router_body,
        grid=(S // _SBLK,),
        in_specs=[
            pl.BlockSpec((_SBLK, H), lambda i: (i, 0)),
            pl.BlockSpec((E, H), lambda i: (0, 0)),
        ],
        out_specs=[
            pl.BlockSpec((_SBLK, E), lambda i: (i, 0)),
            pl.BlockSpec((_SBLK, TOPK), lambda i: (i, 0)),
            pl.BlockSpec((_SBLK, TOPK), lambda i: (i, 0)),
            pl.BlockSpec((_SBLK, TOPK), lambda i: (i, 0)),
            pl.BlockSpec((1, E), lambda i: (0, 0)),
        ],
        out_shape=[
            jax.ShapeDtypeStruct((S, E), jnp.float32),
            jax.ShapeDtypeStruct((S, TOPK), jnp.int32),
            jax.ShapeDtypeStruct((S, TOPK), jnp.float32),
            jax.ShapeDtypeStruct((S, TOPK), jnp.int32),
            jax.ShapeDtypeStruct((1, E), jnp.int32),
        ],
        scratch_shapes=[pltpu.VMEM((1, E), jnp.float32)],
    )(x, wg)


def _mlp_body(te_ref, nr_ref, x_ref, prt_ref, w1_ref, w3_ref, w2_ref, y_ref):
    i = pl.program_id(0)

    @pl.when(i < nr_ref[0])
    def _():
        rowidx = i * TROW + lax.broadcasted_iota(jnp.int32, (TROW, S), 0)
        p0 = prt_ref[0:1, :]
        p1 = prt_ref[1:2, :]
        oh = ((p0 == rowidx) | (p1 == rowidx)).astype(jnp.bfloat16)
        xb = lax.dot_general(oh, x_ref[...], (((1,), (0,)), ((), ())),
                             preferred_element_type=jnp.float32
                             ).astype(jnp.bfloat16)
        a = lax.dot_general(xb, w1_ref[0].astype(jnp.bfloat16),
                            (((1,), (1,)), ((), ())),
                            preferred_element_type=jnp.float32)
        b = lax.dot_general(xb, w3_ref[0].astype(jnp.bfloat16),
                            (((1,), (1,)), ((), ())),
                            preferred_element_type=jnp.float32)
        h = (a * jax.nn.sigmoid(a)) * b
        y_ref[...] = lax.dot_general(h.astype(jnp.bfloat16),
                                     w2_ref[0].astype(jnp.bfloat16),
                                     (((1,), (1,)), ((), ())),
                                     preferred_element_type=jnp.float32)


def _moe_mlp(x16, prowT, w1, w3, w2, tile_expert, n_real):
    def we_idx(i, te, nr):
        return jnp.minimum(te[i], te[jnp.minimum(nr[0] - 1, MAX_TILES - 1)])

    grid_spec = pltpu.PrefetchScalarGridSpec(
        num_scalar_prefetch=2,
        grid=(MAX_TILES,),
        in_specs=[
            pl.BlockSpec((S, H), lambda i, te, nr: (0, 0)),
            pl.BlockSpec((TOPK, S), lambda i, te, nr: (0, 0)),
            pl.BlockSpec((1, F, H), lambda i, te, nr: (we_idx(i, te, nr), 0, 0)),
            pl.BlockSpec((1, F, H), lambda i, te, nr: (we_idx(i, te, nr), 0, 0)),
            pl.BlockSpec((1, H, F), lambda i, te, nr: (we_idx(i, te, nr), 0, 0)),
        ],
        out_specs=pl.BlockSpec((TROW, H), lambda i, te, nr: (i, 0)),
    )
    return pl.pallas_call(
        _mlp_body,
        grid_spec=grid_spec,
        out_shape=jax.ShapeDtypeStruct((MAX_ROWS, H), jnp.float32),
        compiler_params=pltpu.CompilerParams(
            dimension_semantics=("arbitrary",)),
    )(tile_expert, n_real, x16, prowT, w1, w3, w2)


_TPW = S // _NW


def _sc_cparams():
    cp = pltpu.CompilerParams()
    if "needs_layout_passes" in pltpu.CompilerParams.__dataclass_fields__:
        cp = dataclasses.replace(cp, needs_layout_passes=False)
    return cp


def _sc_combine(y, pos0, pos1, w_flat):
    @functools.partial(
        pl.kernel,
        out_type=jax.ShapeDtypeStruct((S, H), jnp.float32),
        mesh=plsc.VectorSubcoreMesh(core_axis_name="c", subcore_axis_name="s"),
        compiler_params=_sc_cparams(),
        scratch_types=[
            pltpu.VMEM((_TPW,), jnp.int32),
            pltpu.VMEM((_TPW,), jnp.int32),
            pltpu.VMEM((2 * _TPW,), jnp.float32),
            pltpu.VMEM((_TPW, H), jnp.float32),
            pltpu.VMEM((_TPW, H), jnp.float32),
            pltpu.SemaphoreType.DMA,
        ],
    )
    def k(y_hbm, p0_hbm, p1_hbm, w_hbm, out_hbm, i0, i1, wv, b0, b1, sem):
        wid = lax.axis_index("s") * _NC + lax.axis_index("c")
        base = wid * _TPW
        pltpu.sync_copy(p0_hbm.at[pl.ds(base, _TPW)], i0)
        pltpu.sync_copy(p1_hbm.at[pl.ds(base, _TPW)], i1)
        pltpu.sync_copy(w_hbm.at[pl.ds(2 * base, 2 * _TPW)], wv)
        pltpu.async_copy(y_hbm.at[i0], b0, sem).wait()
        pltpu.async_copy(y_hbm.at[i1], b1, sem).wait()

        @pl.loop(0, _TPW)
        def _(r):
            w0 = plsc.load_gather(wv, [jnp.full((16,), 2 * r, jnp.int32)])
            w1 = plsc.load_gather(wv, [jnp.full((16,), 2 * r + 1, jnp.int32)])

            @pl.loop(0, H, step=16)
            def _(c):
                b0[r, pl.ds(c, 16)] = (b0[r, pl.ds(c, 16)] * w0 +
                                       b1[r, pl.ds(c, 16)] * w1)

        pltpu.sync_copy(b0, out_hbm.at[pl.ds(base, _TPW)])

    return k(y, pos0, pos1, w_flat)


def kernel(hidden_states, Wg, W1, W3, W2):
    x = hidden_states.reshape(S, H)
    logits, sel, wts, rank, counts = _router(x, Wg)

    counts = counts.reshape(E)
    n_tiles = (counts + TROW - 1) // TROW
    cum_tiles = jnp.cumsum(n_tiles)
    row_base = (cum_tiles - n_tiles) * TROW
    prow2 = (row_base[sel] + rank).astype(jnp.int32)
    n_real = cum_tiles[-1:].astype(jnp.int32)
    tile_expert = jnp.minimum(
        jnp.searchsorted(cum_tiles, jnp.arange(MAX_TILES), side="right"),
        E - 1).astype(jnp.int32)

    w_flat = wts.reshape(A)

    y = _moe_mlp(x.astype(jnp.bfloat16), prow2.T, W1, W3, W2,
                 tile_expert, n_real)
    final = _sc_combine(y, prow2[:, 0], prow2[:, 1], w_flat)
    return final.reshape(hidden_states.shape), logits


# speedup vs baseline: 1.8701x; 1.0863x over previous
import dataclasses
import functools

import jax
import jax.numpy as jnp
from jax import lax
from jax.experimental import pallas as pl
from jax.experimental.pallas import tpu as pltpu
from jax.experimental.pallas import tpu_sc as plsc

S = 2048
H = 768
F = 2048
E = 64
TOPK = 2
A = S * TOPK
TROW = 128
MAX_TILES = 96
MAX_ROWS = MAX_TILES * TROW

_NC = 2
_NS = 16
_NW = _NC * _NS
_SBLK = 256


_NBLK = S // _SBLK


def _router_body(x_ref, wg_ref, lg_ref, wt_ref, prow_ref, te_ref, nr_ref,
                 cnt_ref, sel_s, rank_s):
    i = pl.program_id(0)

    @pl.when(i == 0)
    def _():
        cnt_ref[...] = jnp.zeros((1, E), jnp.float32)

    @pl.when(i < _NBLK)
    def _():
        lg = lax.dot_general(x_ref[...], wg_ref[...], (((1,), (1,)), ((), ())),
                             preferred_element_type=jnp.float32)
        lg_ref[...] = lg
        iota = lax.broadcasted_iota(jnp.int32, lg.shape, 1)
        m1 = jnp.max(lg, axis=1, keepdims=True)
        e1 = jnp.min(jnp.where(lg == m1, iota, E), axis=1, keepdims=True)
        lg2 = jnp.where(iota == e1, -jnp.inf, lg)
        m2 = jnp.max(lg2, axis=1, keepdims=True)
        e2 = jnp.min(jnp.where(lg2 == m2, iota, E), axis=1, keepdims=True)
        w1 = 1.0 / (1.0 + jnp.exp(m2 - m1))
        wt_ref[...] = jnp.concatenate([w1, 1.0 - w1], axis=1)
        sel_s[pl.ds(i * _SBLK, _SBLK), :] = jnp.concatenate([e1, e2], axis=1)

        oh0 = (iota == e1).astype(jnp.bfloat16)
        oh1 = (iota == e2).astype(jnp.bfloat16)
        rr = lax.broadcasted_iota(jnp.int32, (_SBLK, _SBLK), 0)
        cc = lax.broadcasted_iota(jnp.int32, (_SBLK, _SBLK), 1)
        ltri = (rr > cc).astype(jnp.bfloat16)
        p0 = lax.dot_general(ltri, oh0, (((1,), (0,)), ((), ())),
                             preferred_element_type=jnp.float32)
        p1 = lax.dot_general(ltri, oh1, (((1,), (0,)), ((), ())),
                             preferred_element_type=jnp.float32)
        oh0f = oh0.astype(jnp.float32)
        oh1f = oh1.astype(jnp.float32)
        s0 = jnp.sum(oh0f, axis=0, keepdims=True)
        cnt = cnt_ref[...]
        rank0 = jnp.sum(oh0f * (p0 + cnt), axis=1, keepdims=True)
        rank1 = jnp.sum(oh1f * (p1 + cnt + s0), axis=1, keepdims=True)
        rank_s[pl.ds(i * _SBLK, _SBLK), :] = jnp.concatenate(
            [rank0, rank1], axis=1)
        cnt_ref[...] = cnt + s0 + jnp.sum(oh1f, axis=0, keepdims=True)

    @pl.when(i == _NBLK)
    def _():
        cnt = cnt_ref[...]
        n_tiles = jnp.floor((cnt + (TROW - 1)) * (1.0 / TROW))
        iot = lax.broadcasted_iota(jnp.int32, (E, E), 0)
        iot2 = lax.broadcasted_iota(jnp.int32, (E, E), 1)
        ut = (iot <= iot2).astype(jnp.bfloat16)
        cum_tiles = lax.dot_general(n_tiles.astype(jnp.bfloat16), ut,
                                    (((1,), (0,)), ((), ())),
                                    preferred_element_type=jnp.float32)
        row_base = (cum_tiles - n_tiles) * TROW
        nr_ref[...] = cum_tiles[:, E - 1:].astype(jnp.int32)
        tt = lax.broadcasted_iota(jnp.int32, (MAX_TILES, E), 0)
        cum_i = cum_tiles.astype(jnp.int32)
        te = jnp.sum((jnp.broadcast_to(cum_i, (MAX_TILES, E)) <= tt)
                     .astype(jnp.int32), axis=1, keepdims=True)
        te_ref[...] = jnp.minimum(te, E - 1)
        sel = sel_s[...]
        iota_e0 = lax.broadcasted_iota(jnp.int32, (S, E), 1)
        rb = jnp.broadcast_to(row_base, (S, E))
        rb0 = jnp.sum(jnp.where(iota_e0 == sel[:, 0:1], rb, 0.0),
                      axis=1, keepdims=True)
        rb1 = jnp.sum(jnp.where(iota_e0 == sel[:, 1:2], rb, 0.0),
                      axis=1, keepdims=True)
        prow_ref[...] = (jnp.concatenate([rb0, rb1], axis=1) +
                         rank_s[...]).astype(jnp.int32)


def _router(x, wg):
    return pl.pallas_call(
        _router_body,
        grid=(_NBLK + 1,),
        in_specs=[
            pl.BlockSpec((_SBLK, H), lambda i: (jnp.minimum(i, _NBLK - 1), 0)),
            pl.BlockSpec((E, H), lambda i: (0, 0)),
        ],
        out_specs=[
            pl.BlockSpec((_SBLK, E), lambda i: (jnp.minimum(i, _NBLK - 1), 0)),
            pl.BlockSpec((_SBLK, TOPK),
                         lambda i: (jnp.minimum(i, _NBLK - 1), 0)),
            pl.BlockSpec((S, TOPK), lambda i: (0, 0)),
            pl.BlockSpec((MAX_TILES, 1), lambda i: (0, 0)),
            pl.BlockSpec((1, 1), lambda i: (0, 0)),
        ],
        out_shape=[
            jax.ShapeDtypeStruct((S, E), jnp.float32),
            jax.ShapeDtypeStruct((S, TOPK), jnp.float32),
            jax.ShapeDtypeStruct((S, TOPK), jnp.int32),
            jax.ShapeDtypeStruct((MAX_TILES, 1), jnp.int32),
            jax.ShapeDtypeStruct((1, 1), jnp.int32),
        ],
        scratch_shapes=[
            pltpu.VMEM((1, E), jnp.float32),
            pltpu.VMEM((S, TOPK), jnp.int32),
            pltpu.VMEM((S, TOPK), jnp.float32),
        ],
    )(x, wg)


def _mlp_body(te_ref, nr_ref, x_ref, pr_ref, w1_ref, w3_ref, w2_ref, y_ref):
    i = pl.program_id(0)

    @pl.when(i < nr_ref[0])
    def _():
        rowidx = i * TROW + lax.broadcasted_iota(jnp.int32, (S, TROW), 1)
        oh = ((pr_ref[:, 0:1] == rowidx) |
              (pr_ref[:, 1:2] == rowidx)).astype(jnp.bfloat16)
        xb = lax.dot_general(oh, x_ref[...], (((0,), (0,)), ((), ())),
                             preferred_element_type=jnp.float32
                             ).astype(jnp.bfloat16)
        a = lax.dot_general(xb, w1_ref[0].astype(jnp.bfloat16),
                            (((1,), (1,)), ((), ())),
                            preferred_element_type=jnp.float32)
        b = lax.dot_general(xb, w3_ref[0].astype(jnp.bfloat16),
                            (((1,), (1,)), ((), ())),
                            preferred_element_type=jnp.float32)
        h = (a * jax.nn.sigmoid(a)) * b
        y_ref[...] = lax.dot_general(h.astype(jnp.bfloat16),
                                     w2_ref[0].astype(jnp.bfloat16),
                                     (((1,), (1,)), ((), ())),
                                     preferred_element_type=jnp.float32)


def _moe_mlp(x16, prowT, w1, w3, w2, tile_expert, n_real):
    def we_idx(i, te, nr):
        return jnp.minimum(te[i], te[jnp.minimum(nr[0] - 1, MAX_TILES - 1)])

    grid_spec = pltpu.PrefetchScalarGridSpec(
        num_scalar_prefetch=2,
        grid=(MAX_TILES,),
        in_specs=[
            pl.BlockSpec((S, H), lambda i, te, nr: (0, 0)),
            pl.BlockSpec((S, TOPK), lambda i, te, nr: (0, 0)),
            pl.BlockSpec((1, F, H), lambda i, te, nr: (we_idx(i, te, nr), 0, 0)),
            pl.BlockSpec((1, F, H), lambda i, te, nr: (we_idx(i, te, nr), 0, 0)),
            pl.BlockSpec((1, H, F), lambda i, te, nr: (we_idx(i, te, nr), 0, 0)),
        ],
        out_specs=pl.BlockSpec((TROW, H), lambda i, te, nr: (i, 0)),
    )
    return pl.pallas_call(
        _mlp_body,
        grid_spec=grid_spec,
        out_shape=jax.ShapeDtypeStruct((MAX_ROWS, H), jnp.float32),
        compiler_params=pltpu.CompilerParams(
            dimension_semantics=("arbitrary",)),
    )(tile_expert, n_real, x16, prowT, w1, w3, w2)


_TPW = S // _NW


def _sc_cparams():
    cp = pltpu.CompilerParams()
    if "needs_layout_passes" in pltpu.CompilerParams.__dataclass_fields__:
        cp = dataclasses.replace(cp, needs_layout_passes=False)
    return cp


def _sc_combine(y, pos0, pos1, w_flat):
    @functools.partial(
        pl.kernel,
        out_type=jax.ShapeDtypeStruct((S, H), jnp.float32),
        mesh=plsc.VectorSubcoreMesh(core_axis_name="c", subcore_axis_name="s"),
        compiler_params=_sc_cparams(),
        scratch_types=[
            pltpu.VMEM((_TPW,), jnp.int32),
            pltpu.VMEM((_TPW,), jnp.int32),
            pltpu.VMEM((2 * _TPW,), jnp.float32),
            pltpu.VMEM((_TPW, H), jnp.float32),
            pltpu.VMEM((_TPW, H), jnp.float32),
            pltpu.SemaphoreType.DMA,
        ],
    )
    def k(y_hbm, p0_hbm, p1_hbm, w_hbm, out_hbm, i0, i1, wv, b0, b1, sem):
        wid = lax.axis_index("s") * _NC + lax.axis_index("c")
        base = wid * _TPW
        pltpu.sync_copy(p0_hbm.at[pl.ds(base, _TPW)], i0)
        pltpu.sync_copy(p1_hbm.at[pl.ds(base, _TPW)], i1)
        pltpu.sync_copy(w_hbm.at[pl.ds(2 * base, 2 * _TPW)], wv)
        pltpu.async_copy(y_hbm.at[i0], b0, sem).wait()
        pltpu.async_copy(y_hbm.at[i1], b1, sem).wait()

        @pl.loop(0, _TPW)
        def _(r):
            w0 = plsc.load_gather(wv, [jnp.full((16,), 2 * r, jnp.int32)])
            w1 = plsc.load_gather(wv, [jnp.full((16,), 2 * r + 1, jnp.int32)])

            @pl.loop(0, H, step=16)
            def _(c):
                b0[r, pl.ds(c, 16)] = (b0[r, pl.ds(c, 16)] * w0 +
                                       b1[r, pl.ds(c, 16)] * w1)

        pltpu.sync_copy(b0, out_hbm.at[pl.ds(base, _TPW)])

    return k(y, pos0, pos1, w_flat)


def kernel(hidden_states, Wg, W1, W3, W2):
    x = hidden_states.reshape(S, H)
    logits, wts, prow2, tile_expert, n_real = _router(x, Wg)

    y = _moe_mlp(x.astype(jnp.bfloat16), prow2, W1, W3, W2,
                 tile_expert.reshape(MAX_TILES), n_real.reshape(1))
    final = _sc_combine(y, prow2[:, 0], prow2[:, 1], wts.reshape(A))
    return final.reshape(hidden_states.shape), logits
